# Initial kernel scaffold; baseline (speedup 1.0000x reference)
import jax
import jax.numpy as jnp
from jax.experimental import pallas as pl


def kernel(x, edge_index, edge_attr, W1, b1, W2, b2, W3, b3, W4, b4, W8, b8):
    raise NotImplementedError("write your pallas kernel here")


# speedup vs baseline: 6.1171x; 6.1171x over previous
import functools

import jax
import jax.numpy as jnp
from jax import lax
from jax.experimental import pallas as pl
from jax.experimental.pallas import tpu as pltpu
from jax.experimental.pallas import tpu_sc as plsc

N = 10000
D = 128
NP = 10240
NC = 2
NS = 16
NW = NC * NS
C = 128
STRIPE = NP // NS

def _sc_mesh():
    return plsc.VectorSubcoreMesh(core_axis_name="c", subcore_axis_name="s",
                                  num_cores=NC, num_subcores=NS)


def _worker_id():
    c = lax.axis_index("c")
    s = lax.axis_index("s")
    return c, s, s * NC + c


def _deg_body(nchunk, col_hbm, ew_hbm, out_hbm, deg_sh, ci, ewv, zb):
    c, s, w = _worker_id()

    def zb_zero(i, _):
        zb[pl.ds(i * 16, 16)] = jnp.zeros((16,), jnp.float32)
        return 0

    lax.fori_loop(0, STRIPE // 16, zb_zero, 0)
    pltpu.sync_copy(zb, deg_sh.at[pl.ds(s * STRIPE, STRIPE)])
    plsc.subcore_barrier()

    def chunk(k, _):
        base = (w * nchunk + k) * C
        pltpu.sync_copy(col_hbm.at[pl.ds(base, C)], ci)
        pltpu.sync_copy(ew_hbm.at[pl.ds(base, C)], ewv)
        pltpu.sync_copy(ewv, deg_sh.at[ci], add=True)
        return 0

    lax.fori_loop(0, nchunk, chunk, 0)
    plsc.subcore_barrier()
    pltpu.sync_copy(deg_sh.at[pl.ds(s * STRIPE, STRIPE)],
                    out_hbm.at[c].at[pl.ds(s * STRIPE, STRIPE)])


def _sc_degree(colp, ewp, nchunk):
    kfn = pl.kernel(
        functools.partial(_deg_body, nchunk),
        out_type=jax.ShapeDtypeStruct((NC, NP), jnp.float32),
        mesh=_sc_mesh(),
        scratch_types=[
            pltpu.VMEM_SHARED((NP,), jnp.float32),
            pltpu.VMEM((C,), jnp.int32),
            pltpu.VMEM((C,), jnp.float32),
            pltpu.VMEM((STRIPE,), jnp.float32),
        ],
    )
    return kfn(colp, ewp)


def _edge_body(nchunk, row_hbm, col_hbm, ew_hbm, y_hbm, out_hbm,
               acc_sh, ri, ci, ewv, rows, zb, sem):
    c, s, w = _worker_id()

    def zb_zero(i, _):
        for j in range(8):
            zb[i, pl.ds(j * 16, 16)] = jnp.zeros((16,), jnp.float32)
        return 0

    lax.fori_loop(0, 64, zb_zero, 0)

    def zcopy(i, _):
        pltpu.sync_copy(zb, acc_sh.at[pl.ds(s * STRIPE + i * 64, 64)])
        return 0

    lax.fori_loop(0, STRIPE // 64, zcopy, 0)
    plsc.subcore_barrier()

    def chunk(k, _):
        base = (w * nchunk + k) * C
        pltpu.sync_copy(row_hbm.at[pl.ds(base, C)], ri)
        pltpu.sync_copy(col_hbm.at[pl.ds(base, C)], ci)
        pltpu.sync_copy(ew_hbm.at[pl.ds(base, C)], ewv)
        pltpu.async_copy(y_hbm.at[ri], rows, sem).wait()

        def group(g, _):
            ew16 = ewv[pl.ds(g * 16, 16)]
            for j in range(16):
                ew_s = ew16[j]
                e = g * 16 + j
                for k in range(8):
                    sl = pl.ds(k * 16, 16)
                    rows[e, sl] = rows[e, sl] * ew_s
            return 0

        lax.fori_loop(0, C // 16, group, 0)
        pltpu.sync_copy(rows, acc_sh.at[ci], add=True)
        return 0

    lax.fori_loop(0, nchunk, chunk, 0)
    plsc.subcore_barrier()

    def ocopy(i, _):
        pltpu.sync_copy(acc_sh.at[pl.ds(s * STRIPE + i * 64, 64)],
                        out_hbm.at[c].at[pl.ds(s * STRIPE + i * 64, 64)])
        return 0

    lax.fori_loop(0, STRIPE // 64, ocopy, 0)


def _sc_aggregate(rowp, colp, ewp, y, nchunk):
    kfn = pl.kernel(
        functools.partial(_edge_body, nchunk),
        out_type=jax.ShapeDtypeStruct((NC, NP, D), jnp.float32),
        mesh=_sc_mesh(),
        scratch_types=[
            pltpu.VMEM_SHARED((NP, D), jnp.float32),
            pltpu.VMEM((C,), jnp.int32),
            pltpu.VMEM((C,), jnp.int32),
            pltpu.VMEM((C,), jnp.float32),
            pltpu.VMEM((C, D), jnp.float32),
            pltpu.VMEM((64, D), jnp.float32),
            pltpu.SemaphoreType.DMA,
        ],
    )
    return kfn(rowp, colp, ewp, y)


_BR = 2000


def _tc_first_body(deg0_ref, deg1_ref, x_ref, w_ref, dinv_ref, y_ref):
    deg = deg0_ref[...] + deg1_ref[...] + 1.0
    dinv = lax.rsqrt(deg)
    dinv_ref[...] = dinv
    xw = jnp.dot(x_ref[...], w_ref[...], preferred_element_type=jnp.float32)
    y_ref[...] = xw * dinv


def _tc_first(deg0, deg1, x, w1):
    grid = N // _BR
    return pl.pallas_call(
        _tc_first_body,
        grid=(grid,),
        in_specs=[
            pl.BlockSpec((_BR, 1), lambda i: (i, 0)),
            pl.BlockSpec((_BR, 1), lambda i: (i, 0)),
            pl.BlockSpec((_BR, D), lambda i: (i, 0)),
            pl.BlockSpec((D, D), lambda i: (0, 0)),
        ],
        out_specs=[
            pl.BlockSpec((_BR, 1), lambda i: (i, 0)),
            pl.BlockSpec((_BR, D), lambda i: (i, 0)),
        ],
        out_shape=[
            jax.ShapeDtypeStruct((N, 1), jnp.float32),
            jax.ShapeDtypeStruct((N, D), jnp.float32),
        ],
    )(deg0, deg1, x, w1)


def _tc_mid_body(relu, has_w, a0_ref, a1_ref, y_ref, dinv_ref, b_ref, w_ref,
                 out_ref):
    h = (a0_ref[...] + a1_ref[...] + y_ref[...]) * dinv_ref[...] + b_ref[...]
    if relu:
        h = jnp.maximum(h, 0.0)
    if has_w:
        h = jnp.dot(h, w_ref[...], preferred_element_type=jnp.float32)
    out_ref[...] = h * dinv_ref[...]


def _tc_mid(a0, a1, y, dinv, b, w, relu, has_w):
    grid = N // _BR
    return pl.pallas_call(
        functools.partial(_tc_mid_body, relu, has_w),
        grid=(grid,),
        in_specs=[
            pl.BlockSpec((_BR, D), lambda i: (i, 0)),
            pl.BlockSpec((_BR, D), lambda i: (i, 0)),
            pl.BlockSpec((_BR, D), lambda i: (i, 0)),
            pl.BlockSpec((_BR, 1), lambda i: (i, 0)),
            pl.BlockSpec((1, D), lambda i: (0, 0)),
            pl.BlockSpec((D, D), lambda i: (0, 0)),
        ],
        out_specs=pl.BlockSpec((_BR, D), lambda i: (i, 0)),
        out_shape=jax.ShapeDtypeStruct((N, D), jnp.float32),
    )(a0, a1, y, dinv, b, w)


def _tc_last_body(a0_ref, a1_ref, y_ref, dinv_ref, w_ref, b_ref, out_ref):
    g = (a0_ref[...] + a1_ref[...] + y_ref[...]) * dinv_ref[...]
    out_ref[...] = (
        jnp.dot(g, w_ref[...], preferred_element_type=jnp.float32)
        + b_ref[...]
    )


def _tc_last(a0, a1, y, dinv, w8, b8):
    grid = N // _BR
    return pl.pallas_call(
        _tc_last_body,
        grid=(grid,),
        in_specs=[
            pl.BlockSpec((_BR, D), lambda i: (i, 0)),
            pl.BlockSpec((_BR, D), lambda i: (i, 0)),
            pl.BlockSpec((_BR, D), lambda i: (i, 0)),
            pl.BlockSpec((_BR, 1), lambda i: (i, 0)),
            pl.BlockSpec((D, 2), lambda i: (0, 0)),
            pl.BlockSpec((1, 2), lambda i: (0, 0)),
        ],
        out_specs=pl.BlockSpec((_BR, 2), lambda i: (i, 0)),
        out_shape=jax.ShapeDtypeStruct((N, 2), jnp.float32),
    )(a0, a1, y, dinv, w8, b8)


def kernel(x, edge_index, edge_attr, W1, b1, W2, b2, W3, b3, W4, b4, W8, b8):
    E = edge_index.shape[1]
    nchunk = -(-E // (NW * C))
    EP = NW * C * nchunk
    pad = EP - E

    row = edge_index[0]
    col = edge_index[1]
    rowp = jnp.concatenate([row, jnp.zeros((pad,), row.dtype)])
    colp = jnp.concatenate([col, jnp.zeros((pad,), col.dtype)])
    ewp = jnp.concatenate([edge_attr, jnp.zeros((pad,), edge_attr.dtype)])

    deg_part = _sc_degree(colp, ewp, nchunk)
    deg0 = deg_part[0, :N].reshape(N, 1)
    deg1 = deg_part[1, :N].reshape(N, 1)

    dinv, y = _tc_first(deg0, deg1, x, W1)

    def agg(yv):
        acc = _sc_aggregate(rowp, colp, ewp, yv, nchunk)
        return acc[0, :N], acc[1, :N]

    a0, a1 = agg(y)
    y = _tc_mid(a0, a1, y, dinv, b1.reshape(1, D), W2, True, True)
    a0, a1 = agg(y)
    y = _tc_mid(a0, a1, y, dinv, b2.reshape(1, D), W3, True, True)
    a0, a1 = agg(y)
    y = _tc_mid(a0, a1, y, dinv, b3.reshape(1, D), W4, True, True)
    a0, a1 = agg(y)
    y = _tc_mid(a0, a1, y, dinv, b4.reshape(1, D), W4, False, False)
    a0, a1 = agg(y)
    return _tc_last(a0, a1, y, dinv, W8, b8.reshape(1, 2))


# speedup vs baseline: 11.0592x; 1.8079x over previous
import functools

import jax
import jax.numpy as jnp
from jax import lax
from jax.experimental import pallas as pl
from jax.experimental.pallas import tpu as pltpu
from jax.experimental.pallas import tpu_sc as plsc

N = 10000
D = 128
NP = 10240
NC = 2
NS = 16
NW = NC * NS
C = 112
STRIPE = NP // NS
STRIPE_A = N // NS

def _sc_mesh():
    return plsc.VectorSubcoreMesh(core_axis_name="c", subcore_axis_name="s",
                                  num_cores=NC, num_subcores=NS)


def _worker_id():
    c = lax.axis_index("c")
    s = lax.axis_index("s")
    return c, s, s * NC + c


def _deg_body(nchunk, col_hbm, ew_hbm, out_hbm, deg_sh, ci, ewv, zb):
    c, s, w = _worker_id()

    def zb_zero(i, _):
        zb[pl.ds(i * 16, 16)] = jnp.zeros((16,), jnp.float32)
        return 0

    lax.fori_loop(0, STRIPE // 16, zb_zero, 0)
    pltpu.sync_copy(zb, deg_sh.at[pl.ds(s * STRIPE, STRIPE)])
    plsc.subcore_barrier()

    def chunk(k, _):
        base = (w * nchunk + k) * C
        pltpu.sync_copy(col_hbm.at[pl.ds(base, C)], ci)
        pltpu.sync_copy(ew_hbm.at[pl.ds(base, C)], ewv)
        pltpu.sync_copy(ewv, deg_sh.at[ci], add=True)
        return 0

    lax.fori_loop(0, nchunk, chunk, 0)
    plsc.subcore_barrier()
    pltpu.sync_copy(deg_sh.at[pl.ds(s * STRIPE, STRIPE)],
                    out_hbm.at[c].at[pl.ds(s * STRIPE, STRIPE)])


def _sc_degree(colp, ewp, nchunk):
    kfn = pl.kernel(
        functools.partial(_deg_body, nchunk),
        out_type=jax.ShapeDtypeStruct((NC, NP), jnp.float32),
        mesh=_sc_mesh(),
        scratch_types=[
            pltpu.VMEM_SHARED((NP,), jnp.float32),
            pltpu.VMEM((C,), jnp.int32),
            pltpu.VMEM((C,), jnp.float32),
            pltpu.VMEM((STRIPE,), jnp.float32),
        ],
    )
    return kfn(colp, ewp)


def _edge_body(nchunk, row_hbm, col_hbm, ew_hbm, y_hbm, out_hbm,
               acc_sh, ri0, ri1, ri2, ci0, ci1, ci2, ew0, ew1, ew2,
               rows0, rows1, rows2,
               gsem0, gsem1, gsem2, isem0, isem1, isem2,
               ssem0, ssem1, ssem2):
    c, s, w = _worker_id()
    ri = (ri0, ri1, ri2)
    ci = (ci0, ci1, ci2)
    ewv = (ew0, ew1, ew2)
    rows = (rows0, rows1, rows2)
    gsem = (gsem0, gsem1, gsem2)
    isem = (isem0, isem1, isem2)
    ssem = (ssem0, ssem1, ssem2)
    base0 = w * nchunk * C

    def zb_zero(i, _):
        for j in range(8):
            rows0[i, pl.ds(j * 16, 16)] = jnp.zeros((16,), jnp.float32)
        return 0

    lax.fori_loop(0, C, zb_zero, 0)

    def zcopy(i, _):
        pltpu.sync_copy(rows0.at[pl.ds(0, 64)],
                        acc_sh.at[pl.ds(s * STRIPE + i * 64, 64)])
        return 0

    lax.fori_loop(0, STRIPE // 64, zcopy, 0)
    plsc.subcore_barrier()

    def idx_fetch(k, b):
        sl = pl.ds(base0 + k * C, C)
        return (pltpu.async_copy(row_hbm.at[sl], ri[b], isem[b]),
                pltpu.async_copy(col_hbm.at[sl], ci[b], isem[b]),
                pltpu.async_copy(ew_hbm.at[sl], ewv[b], isem[b]))

    def gather_start(b):
        pltpu.async_copy(y_hbm.at[ri[b]], rows[b], gsem[b])

    def gather_wait(b):
        pltpu.make_async_copy(y_hbm.at[ri[b]], rows[b], gsem[b]).wait()

    def scale_scatter(b):
        def group(g, _):
            ew16 = ewv[b][pl.ds(g * 16, 16)]
            for j in range(16):
                ew_s = ew16[j]
                e = g * 16 + j
                for q in range(8):
                    sl = pl.ds(q * 16, 16)
                    rows[b][e, sl] = rows[b][e, sl] * ew_s
            return 0

        lax.fori_loop(0, C // 16, group, 0)
        pltpu.async_copy(rows[b], acc_sh.at[ci[b]], ssem[b], add=True)

    def scatter_wait(b):
        pltpu.make_async_copy(rows[b], acc_sh.at[ci[b]], ssem[b]).wait()

    def step(k, b, wait_scatter):
        b1 = (b + 1) % 3
        if wait_scatter:
            scatter_wait(b1)
        for dsc in idx_fetch(k + 1, b1):
            dsc.wait()
        gather_start(b1)
        gather_wait(b)
        scale_scatter(b)

    for dsc in idx_fetch(0, 0):
        dsc.wait()
    gather_start(0)
    step(0, 0, False)
    step(1, 1, False)

    @pl.loop(2, nchunk - 1, step=3)
    def _triple(k):
        step(k, 2, True)
        step(k + 1, 0, True)
        step(k + 2, 1, True)

    gather_wait(2)
    scale_scatter(2)
    scatter_wait(0)
    scatter_wait(1)
    scatter_wait(2)

    plsc.subcore_barrier()

    def ocopy(i, _):
        pltpu.sync_copy(acc_sh.at[pl.ds(s * STRIPE + i * 64, 64)],
                        out_hbm.at[c].at[pl.ds(s * STRIPE + i * 64, 64)])
        return 0

    lax.fori_loop(0, STRIPE // 64, ocopy, 0)


def _sc_aggregate(rowp, colp, ewp, y, nchunk):
    kfn = pl.kernel(
        functools.partial(_edge_body, nchunk),
        out_type=jax.ShapeDtypeStruct((NC, NP, D), jnp.float32),
        mesh=_sc_mesh(),
        scratch_types=(
            [pltpu.VMEM_SHARED((NP, D), jnp.float32)]
            + [pltpu.VMEM((C,), jnp.int32)] * 6
            + [pltpu.VMEM((C,), jnp.float32)] * 3
            + [pltpu.VMEM((C, D), jnp.float32)] * 3
            + [pltpu.SemaphoreType.DMA] * 9
        ),
    )
    return kfn(rowp, colp, ewp, y)


_BR = 2000


def _tc_first_body(deg0_ref, deg1_ref, x_ref, w_ref, dinv_ref, y_ref):
    deg = deg0_ref[...] + deg1_ref[...] + 1.0
    dinv = lax.rsqrt(deg)
    dinv_ref[...] = dinv
    xw = jnp.dot(x_ref[...], w_ref[...], preferred_element_type=jnp.float32)
    y_ref[...] = xw * dinv


def _tc_first(deg0, deg1, x, w1):
    grid = N // _BR
    return pl.pallas_call(
        _tc_first_body,
        grid=(grid,),
        in_specs=[
            pl.BlockSpec((_BR, 1), lambda i: (i, 0)),
            pl.BlockSpec((_BR, 1), lambda i: (i, 0)),
            pl.BlockSpec((_BR, D), lambda i: (i, 0)),
            pl.BlockSpec((D, D), lambda i: (0, 0)),
        ],
        out_specs=[
            pl.BlockSpec((_BR, 1), lambda i: (i, 0)),
            pl.BlockSpec((_BR, D), lambda i: (i, 0)),
        ],
        out_shape=[
            jax.ShapeDtypeStruct((N, 1), jnp.float32),
            jax.ShapeDtypeStruct((N, D), jnp.float32),
        ],
    )(deg0, deg1, x, w1)


def _tc_mid_body(relu, has_w, a0_ref, a1_ref, y_ref, dinv_ref, b_ref, w_ref,
                 out_ref):
    h = (a0_ref[...] + a1_ref[...] + y_ref[...]) * dinv_ref[...] + b_ref[...]
    if relu:
        h = jnp.maximum(h, 0.0)
    if has_w:
        h = jnp.dot(h, w_ref[...], preferred_element_type=jnp.float32)
    out_ref[...] = h * dinv_ref[...]


def _tc_mid(a0, a1, y, dinv, b, w, relu, has_w):
    grid = N // _BR
    return pl.pallas_call(
        functools.partial(_tc_mid_body, relu, has_w),
        grid=(grid,),
        in_specs=[
            pl.BlockSpec((_BR, D), lambda i: (i, 0)),
            pl.BlockSpec((_BR, D), lambda i: (i, 0)),
            pl.BlockSpec((_BR, D), lambda i: (i, 0)),
            pl.BlockSpec((_BR, 1), lambda i: (i, 0)),
            pl.BlockSpec((1, D), lambda i: (0, 0)),
            pl.BlockSpec((D, D), lambda i: (0, 0)),
        ],
        out_specs=pl.BlockSpec((_BR, D), lambda i: (i, 0)),
        out_shape=jax.ShapeDtypeStruct((N, D), jnp.float32),
    )(a0, a1, y, dinv, b, w)


def _tc_last_body(a0_ref, a1_ref, y_ref, dinv_ref, w_ref, b_ref, out_ref):
    g = (a0_ref[...] + a1_ref[...] + y_ref[...]) * dinv_ref[...]
    out_ref[...] = (
        jnp.dot(g, w_ref[...], preferred_element_type=jnp.float32)
        + b_ref[...]
    )


def _tc_last(a0, a1, y, dinv, w8, b8):
    grid = N // _BR
    return pl.pallas_call(
        _tc_last_body,
        grid=(grid,),
        in_specs=[
            pl.BlockSpec((_BR, D), lambda i: (i, 0)),
            pl.BlockSpec((_BR, D), lambda i: (i, 0)),
            pl.BlockSpec((_BR, D), lambda i: (i, 0)),
            pl.BlockSpec((_BR, 1), lambda i: (i, 0)),
            pl.BlockSpec((D, 2), lambda i: (0, 0)),
            pl.BlockSpec((1, 2), lambda i: (0, 0)),
        ],
        out_specs=pl.BlockSpec((_BR, 2), lambda i: (i, 0)),
        out_shape=jax.ShapeDtypeStruct((N, 2), jnp.float32),
    )(a0, a1, y, dinv, w8, b8)


def kernel(x, edge_index, edge_attr, W1, b1, W2, b2, W3, b3, W4, b4, W8, b8):
    E = edge_index.shape[1]
    nchunk = -(-E // (NW * C))
    nchunk += (-nchunk) % 3
    EP = NW * C * nchunk
    pad = EP - E

    row = edge_index[0]
    col = edge_index[1]
    rowp = jnp.concatenate([row, jnp.zeros((pad,), row.dtype)])
    colp = jnp.concatenate([col, jnp.zeros((pad,), col.dtype)])
    ewp = jnp.concatenate([edge_attr, jnp.zeros((pad,), edge_attr.dtype)])

    deg_part = _sc_degree(colp, ewp, nchunk)
    deg0 = deg_part[0, :N].reshape(N, 1)
    deg1 = deg_part[1, :N].reshape(N, 1)

    dinv, y = _tc_first(deg0, deg1, x, W1)

    def agg(yv):
        acc = _sc_aggregate(rowp, colp, ewp, yv, nchunk)
        return acc[0, :N], acc[1, :N]

    a0, a1 = agg(y)
    y = _tc_mid(a0, a1, y, dinv, b1.reshape(1, D), W2, True, True)
    a0, a1 = agg(y)
    y = _tc_mid(a0, a1, y, dinv, b2.reshape(1, D), W3, True, True)
    a0, a1 = agg(y)
    y = _tc_mid(a0, a1, y, dinv, b3.reshape(1, D), W4, True, True)
    a0, a1 = agg(y)
    y = _tc_mid(a0, a1, y, dinv, b4.reshape(1, D), W4, False, False)
    a0, a1 = agg(y)
    return _tc_last(a0, a1, y, dinv, W8, b8.reshape(1, 2))


# speedup vs baseline: 12.2916x; 1.1114x over previous
import functools

import jax
import jax.numpy as jnp
from jax import lax
from jax.experimental import pallas as pl
from jax.experimental.pallas import tpu as pltpu
from jax.experimental.pallas import tpu_sc as plsc

N = 10000
D = 128
NP = 10240
NC = 2
NS = 16
NW = NC * NS
C = 112
NCK_A = 117
NCK_B = 63
STRIPE = NP // NS
STRIPE_A = N // NS

def _sc_mesh():
    return plsc.VectorSubcoreMesh(core_axis_name="c", subcore_axis_name="s",
                                  num_cores=NC, num_subcores=NS)


def _worker_id():
    c = lax.axis_index("c")
    s = lax.axis_index("s")
    return c, s, s * NC + c


def _deg_body(nchunk, col_hbm, ew_hbm, out_hbm, deg_sh, ci, ewv, zb):
    c, s, w = _worker_id()

    def zb_zero(i, _):
        zb[pl.ds(i * 16, 16)] = jnp.zeros((16,), jnp.float32)
        return 0

    lax.fori_loop(0, STRIPE // 16, zb_zero, 0)
    pltpu.sync_copy(zb, deg_sh.at[pl.ds(s * STRIPE, STRIPE)])
    plsc.subcore_barrier()

    def chunk(k, _):
        base = (w * nchunk + k) * C
        pltpu.sync_copy(col_hbm.at[pl.ds(base, C)], ci)
        pltpu.sync_copy(ew_hbm.at[pl.ds(base, C)], ewv)
        pltpu.sync_copy(ewv, deg_sh.at[ci], add=True)
        return 0

    lax.fori_loop(0, nchunk, chunk, 0)
    plsc.subcore_barrier()
    pltpu.sync_copy(deg_sh.at[pl.ds(s * STRIPE, STRIPE)],
                    out_hbm.at[c].at[pl.ds(s * STRIPE, STRIPE)])


def _sc_degree(colp, ewp, nchunk):
    kfn = pl.kernel(
        functools.partial(_deg_body, nchunk),
        out_type=jax.ShapeDtypeStruct((NC, NP), jnp.float32),
        mesh=_sc_mesh(),
        scratch_types=[
            pltpu.VMEM_SHARED((NP,), jnp.float32),
            pltpu.VMEM((C,), jnp.int32),
            pltpu.VMEM((C,), jnp.float32),
            pltpu.VMEM((STRIPE,), jnp.float32),
        ],
    )
    return kfn(colp, ewp)


def _edge_body(row_hbm, col_hbm, ew_hbm, y_hbm, out_hbm,
               acc_sh, ri0, ri1, ri2, ci0, ci1, ci2, ew0, ew1, ew2,
               rows0, rows1, rows2,
               gsem0, gsem1, gsem2, isem0, isem1, isem2,
               ssem0, ssem1, ssem2):
    c, s, w = _worker_id()
    ri = (ri0, ri1, ri2)
    ci = (ci0, ci1, ci2)
    ewv = (ew0, ew1, ew2)
    rows = (rows0, rows1, rows2)
    gsem = (gsem0, gsem1, gsem2)
    isem = (isem0, isem1, isem2)
    ssem = (ssem0, ssem1, ssem2)
    nck = jnp.where(c == 0, NCK_A, NCK_B)
    base0 = jnp.where(c == 0, s * (NCK_A * C),
                      NS * (NCK_A * C) + s * (NCK_B * C))

    def zb_zero(i, _):
        for j in range(8):
            rows0[i, pl.ds(j * 16, 16)] = jnp.zeros((16,), jnp.float32)
        return 0

    lax.fori_loop(0, C, zb_zero, 0)

    zdescs = [
        pltpu.async_copy(rows0.at[pl.ds(0, 64)],
                         acc_sh.at[pl.ds(s * STRIPE + i * 64, 64)],
                         gsem0)
        for i in range(STRIPE // 64)
    ]
    for dsc in zdescs:
        dsc.wait()
    plsc.subcore_barrier()

    def idx_fetch(k, b):
        sl = pl.ds(base0 + k * C, C)
        return (pltpu.async_copy(row_hbm.at[sl], ri[b], isem[b]),
                pltpu.async_copy(col_hbm.at[sl], ci[b], isem[b]),
                pltpu.async_copy(ew_hbm.at[sl], ewv[b], isem[b]))

    def gather_start(b):
        pltpu.async_copy(y_hbm.at[ri[b]], rows[b], gsem[b])

    def gather_wait(b):
        pltpu.make_async_copy(y_hbm.at[ri[b]], rows[b], gsem[b]).wait()

    def scale_scatter(b):
        @plsc.parallel_loop(0, C // 16, unroll=1)
        def group(g):
            ew16 = ewv[b][pl.ds(g * 16, 16)]
            for j in range(16):
                ew_s = ew16[j]
                e = g * 16 + j
                for q in range(8):
                    sl = pl.ds(q * 16, 16)
                    rows[b][e, sl] = rows[b][e, sl] * ew_s

        pltpu.async_copy(rows[b], acc_sh.at[ci[b]], ssem[b], add=True)

    def scatter_wait(b):
        pltpu.make_async_copy(rows[b], acc_sh.at[ci[b]], ssem[b]).wait()

    def step(k, b, wait_scatter):
        b1 = (b + 1) % 3
        if wait_scatter:
            scatter_wait(b1)
        for dsc in idx_fetch(k + 1, b1):
            dsc.wait()
        gather_start(b1)
        gather_wait(b)
        scale_scatter(b)

    for dsc in idx_fetch(0, 0):
        dsc.wait()
    gather_start(0)
    step(0, 0, False)
    step(1, 1, False)

    @pl.loop(2, nck - 1, step=3)
    def _triple(k):
        step(k, 2, True)
        step(k + 1, 0, True)
        step(k + 2, 1, True)

    gather_wait(2)
    scale_scatter(2)
    scatter_wait(0)
    scatter_wait(1)
    scatter_wait(2)

    plsc.subcore_barrier()

    odescs = [
        pltpu.async_copy(acc_sh.at[pl.ds(s * STRIPE + i * 64, 64)],
                         out_hbm.at[c].at[pl.ds(s * STRIPE + i * 64, 64)],
                         gsem0)
        for i in range(STRIPE // 64)
    ]
    for dsc in odescs:
        dsc.wait()


def _sc_aggregate(rowp, colp, ewp, y):
    kfn = pl.kernel(
        _edge_body,
        out_type=jax.ShapeDtypeStruct((NC, NP, D), jnp.float32),
        mesh=_sc_mesh(),
        scratch_types=(
            [pltpu.VMEM_SHARED((NP, D), jnp.float32)]
            + [pltpu.VMEM((C,), jnp.int32)] * 6
            + [pltpu.VMEM((C,), jnp.float32)] * 3
            + [pltpu.VMEM((C, D), jnp.float32)] * 3
            + [pltpu.SemaphoreType.DMA] * 9
        ),
    )
    return kfn(rowp, colp, ewp, y)


_BR = 2000


def _tc_first_body(deg0_ref, deg1_ref, x_ref, w_ref, dinv_ref, y_ref):
    deg = deg0_ref[...] + deg1_ref[...] + 1.0
    dinv = lax.rsqrt(deg)
    dinv_ref[...] = dinv
    xw = jnp.dot(x_ref[...], w_ref[...], preferred_element_type=jnp.float32)
    y_ref[...] = xw * dinv


def _tc_first(deg0, deg1, x, w1):
    grid = N // _BR
    return pl.pallas_call(
        _tc_first_body,
        grid=(grid,),
        in_specs=[
            pl.BlockSpec((_BR, 1), lambda i: (i, 0)),
            pl.BlockSpec((_BR, 1), lambda i: (i, 0)),
            pl.BlockSpec((_BR, D), lambda i: (i, 0)),
            pl.BlockSpec((D, D), lambda i: (0, 0)),
        ],
        out_specs=[
            pl.BlockSpec((_BR, 1), lambda i: (i, 0)),
            pl.BlockSpec((_BR, D), lambda i: (i, 0)),
        ],
        out_shape=[
            jax.ShapeDtypeStruct((N, 1), jnp.float32),
            jax.ShapeDtypeStruct((N, D), jnp.float32),
        ],
    )(deg0, deg1, x, w1)


def _tc_mid_body(relu, has_w, a0_ref, a1_ref, y_ref, dinv_ref, b_ref, w_ref,
                 out_ref):
    h = (a0_ref[...] + a1_ref[...] + y_ref[...]) * dinv_ref[...] + b_ref[...]
    if relu:
        h = jnp.maximum(h, 0.0)
    if has_w:
        h = jnp.dot(h, w_ref[...], preferred_element_type=jnp.float32)
    out_ref[...] = h * dinv_ref[...]


def _tc_mid(a0, a1, y, dinv, b, w, relu, has_w):
    grid = N // _BR
    return pl.pallas_call(
        functools.partial(_tc_mid_body, relu, has_w),
        grid=(grid,),
        in_specs=[
            pl.BlockSpec((_BR, D), lambda i: (i, 0)),
            pl.BlockSpec((_BR, D), lambda i: (i, 0)),
            pl.BlockSpec((_BR, D), lambda i: (i, 0)),
            pl.BlockSpec((_BR, 1), lambda i: (i, 0)),
            pl.BlockSpec((1, D), lambda i: (0, 0)),
            pl.BlockSpec((D, D), lambda i: (0, 0)),
        ],
        out_specs=pl.BlockSpec((_BR, D), lambda i: (i, 0)),
        out_shape=jax.ShapeDtypeStruct((N, D), jnp.float32),
    )(a0, a1, y, dinv, b, w)


def _tc_last_body(a0_ref, a1_ref, y_ref, dinv_ref, w_ref, b_ref, out_ref):
    g = (a0_ref[...] + a1_ref[...] + y_ref[...]) * dinv_ref[...]
    out_ref[...] = (
        jnp.dot(g, w_ref[...], preferred_element_type=jnp.float32)
        + b_ref[...]
    )


def _tc_last(a0, a1, y, dinv, w8, b8):
    grid = N // _BR
    return pl.pallas_call(
        _tc_last_body,
        grid=(grid,),
        in_specs=[
            pl.BlockSpec((_BR, D), lambda i: (i, 0)),
            pl.BlockSpec((_BR, D), lambda i: (i, 0)),
            pl.BlockSpec((_BR, D), lambda i: (i, 0)),
            pl.BlockSpec((_BR, 1), lambda i: (i, 0)),
            pl.BlockSpec((D, 2), lambda i: (0, 0)),
            pl.BlockSpec((1, 2), lambda i: (0, 0)),
        ],
        out_specs=pl.BlockSpec((_BR, 2), lambda i: (i, 0)),
        out_shape=jax.ShapeDtypeStruct((N, 2), jnp.float32),
    )(a0, a1, y, dinv, w8, b8)


def kernel(x, edge_index, edge_attr, W1, b1, W2, b2, W3, b3, W4, b4, W8, b8):
    E = edge_index.shape[1]
    EP = NS * (NCK_A + NCK_B) * C
    assert EP >= E
    pad = EP - E
    nchunk_deg = EP // (NW * C)

    row = edge_index[0]
    col = edge_index[1]
    rowp = jnp.concatenate([row, jnp.zeros((pad,), row.dtype)])
    colp = jnp.concatenate([col, jnp.zeros((pad,), col.dtype)])
    ewp = jnp.concatenate([edge_attr, jnp.zeros((pad,), edge_attr.dtype)])

    deg_part = _sc_degree(colp, ewp, nchunk_deg)
    deg0 = deg_part[0, :N].reshape(N, 1)
    deg1 = deg_part[1, :N].reshape(N, 1)

    dinv, y = _tc_first(deg0, deg1, x, W1)

    def agg(yv):
        acc = _sc_aggregate(rowp, colp, ewp, yv)
        return acc[0, :N], acc[1, :N]

    a0, a1 = agg(y)
    y = _tc_mid(a0, a1, y, dinv, b1.reshape(1, D), W2, True, True)
    a0, a1 = agg(y)
    y = _tc_mid(a0, a1, y, dinv, b2.reshape(1, D), W3, True, True)
    a0, a1 = agg(y)
    y = _tc_mid(a0, a1, y, dinv, b3.reshape(1, D), W4, True, True)
    a0, a1 = agg(y)
    y = _tc_mid(a0, a1, y, dinv, b4.reshape(1, D), W4, False, False)
    a0, a1 = agg(y)
    return _tc_last(a0, a1, y, dinv, W8, b8.reshape(1, 2))
